# baseline (speedup 1.0000x reference)
import functools

import jax
import jax.numpy as jnp
from jax import lax
from jax.experimental import pallas as pl
from jax.experimental.pallas import tpu as pltpu
from jax.experimental.pallas import tpu_sc as plsc

N = 10000
NPAD = 10240
D = 128
DH = 64
E = 320000
B = 125
NBLK = E // B
NSUB = 16
NB = NBLK // NSUB
IDXC = 40
NSLAB = NB // IDXC
ROWS_PER_TILE = NPAD // NSUB
CW = 16
DIV_CHUNK = 40
NBUF = 8
AHEAD = 6


def _sc_scatter_mean(srcb, dstb, h0, h1):
  mesh = plsc.VectorSubcoreMesh(core_axis_name="c", subcore_axis_name="s")

  @functools.partial(
      pl.kernel,
      out_type=jax.ShapeDtypeStruct((NPAD, D), jnp.float32),
      mesh=mesh,
      scratch_types=[
          pltpu.VMEM_SHARED((NPAD, DH), jnp.float32),
          pltpu.VMEM_SHARED((NPAD, CW), jnp.float32),
          pltpu.VMEM((IDXC, B), jnp.int32),
          pltpu.VMEM((IDXC, B), jnp.int32),
          pltpu.VMEM((NBUF, B, DH), jnp.float32),
          pltpu.VMEM((B, CW), jnp.float32),
          pltpu.VMEM((DIV_CHUNK, DH), jnp.float32),
          pltpu.VMEM((DIV_CHUNK, CW), jnp.float32),
          pltpu.SemaphoreType.DMA,
          pltpu.SemaphoreType.DMA,
          pltpu.SemaphoreType.DMA,
          pltpu.SemaphoreType.DMA,
          pltpu.SemaphoreType.DMA,
          pltpu.SemaphoreType.DMA,
          pltpu.SemaphoreType.DMA,
          pltpu.SemaphoreType.DMA,
          pltpu.SemaphoreType.DMA,
          pltpu.SemaphoreType.DMA,
          pltpu.SemaphoreType.DMA,
          pltpu.SemaphoreType.DMA,
          pltpu.SemaphoreType.DMA,
          pltpu.SemaphoreType.DMA,
          pltpu.SemaphoreType.DMA,
          pltpu.SemaphoreType.DMA,
          pltpu.SemaphoreType.DMA,
          pltpu.SemaphoreType.DMA,
          pltpu.SemaphoreType.DMA,
          pltpu.SemaphoreType.DMA,
          pltpu.SemaphoreType.DMA,
          pltpu.SemaphoreType.DMA,
          pltpu.SemaphoreType.DMA,
          pltpu.SemaphoreType.DMA,
      ],
      compiler_params=pltpu.CompilerParams(use_tc_tiling_on_sc=False),
  )
  def k(srcb_hbm, dstb_hbm, h0_hbm, h1_hbm, out_hbm,
        acc, cnt, src_v, dst_v, rows, ones_v, accv, cntv,
        g0, g1, g2, g3, g4, g5, g6, g7, s0, s1, s2, s3, s4, s5, s6, s7,
        o0, o1, o2, o3, o4, o5, o6, o7):
    c = lax.axis_index("c")
    s = lax.axis_index("s")
    row0 = s * ROWS_PER_TILE
    blk0 = s * NB
    gsem = [g0, g1, g2, g3, g4, g5, g6, g7]
    ssem = [s0, s1, s2, s3, s4, s5, s6, s7]
    osem = [o0, o1, o2, o3, o4, o5, o6, o7]

    @pl.loop(0, B)
    def _(i):
      ones_v[i, :] = jnp.ones((CW,), jnp.float32)

    @pl.loop(0, DIV_CHUNK)
    def _(i):
      cntv[i, :] = jnp.zeros((CW,), jnp.float32)
      for q in range(DH // 16):
        accv[i, pl.ds(q * 16, 16)] = jnp.zeros((16,), jnp.float32)

    @pl.loop(0, ROWS_PER_TILE, step=DIV_CHUNK)
    def _(t):
      pltpu.sync_copy(accv, acc.at[pl.ds(row0 + t, DIV_CHUNK)])
      pltpu.sync_copy(cntv, cnt.at[pl.ds(row0 + t, DIV_CHUNK)])

    plsc.subcore_barrier()

    def gather(j, b):
      @pl.when(c == 0)
      def _():
        pltpu.async_copy(h0_hbm.at[src_v.at[j]], rows.at[b], gsem[b])

      @pl.when(c == 1)
      def _():
        pltpu.async_copy(h1_hbm.at[src_v.at[j]], rows.at[b], gsem[b])

    def gather_wait(j, b):
      pltpu.make_async_copy(h0_hbm.at[src_v.at[j]], rows.at[b], gsem[b]).wait()

    @pl.loop(0, NSLAB)
    def _(sl):
      sblk = blk0 + sl * IDXC
      pltpu.async_copy(srcb_hbm.at[pl.ds(sblk, IDXC)], src_v, g0)
      pltpu.async_copy(dstb_hbm.at[pl.ds(sblk, IDXC)], dst_v, g1)
      pltpu.make_async_copy(srcb_hbm.at[pl.ds(sblk, IDXC)], src_v, g0).wait()
      pltpu.make_async_copy(dstb_hbm.at[pl.ds(sblk, IDXC)], dst_v, g1).wait()

      for j in range(AHEAD):
        gather(j, j)

      @pl.loop(0, IDXC, step=NBUF)
      def _(i):
        for r in range(NBUF):
          b = r
          jj = i + r
          gather_wait(jj, b)
          pltpu.async_copy(rows.at[b], acc.at[dst_v.at[jj]], ssem[b], add=True)
          pltpu.async_copy(ones_v, cnt.at[dst_v.at[jj]], osem[b], add=True)

          @pl.when(jj >= NBUF - AHEAD)
          def _():
            bw = (r + AHEAD) % NBUF
            pltpu.make_async_copy(rows.at[bw],
                                  acc.at[dst_v.at[jj - (NBUF - AHEAD)]],
                                  ssem[bw]).wait()
            pltpu.make_async_copy(ones_v,
                                  cnt.at[dst_v.at[jj - (NBUF - AHEAD)]],
                                  osem[bw]).wait()

          @pl.when(jj + AHEAD < IDXC)
          def _():
            gather(jj + AHEAD, (r + AHEAD) % NBUF)

      for jj in range(IDXC - (NBUF - AHEAD), IDXC):
        b = jj % NBUF
        pltpu.make_async_copy(rows.at[b], acc.at[dst_v.at[jj]], ssem[b]).wait()
        pltpu.make_async_copy(ones_v, cnt.at[dst_v.at[jj]], osem[b]).wait()

    plsc.subcore_barrier()

    @pl.loop(0, ROWS_PER_TILE, step=DIV_CHUNK)
    def _(t):
      pltpu.sync_copy(acc.at[pl.ds(row0 + t, DIV_CHUNK)], accv)
      pltpu.sync_copy(cnt.at[pl.ds(row0 + t, DIV_CHUNK)], cntv)

      @pl.loop(0, DIV_CHUNK)
      def _(i):
        r = 1.0 / jnp.maximum(cntv[i, :], 1.0)
        for q in range(DH // 16):
          accv[i, pl.ds(q * 16, 16)] = accv[i, pl.ds(q * 16, 16)] * r

      pltpu.sync_copy(
          accv, out_hbm.at[pl.ds(row0 + t, DIV_CHUNK), pl.ds(c * DH, DH)])

  return k(srcb, dstb, h0, h1)


@jax.jit
def kernel(edge_index, h):
  src = edge_index[0].astype(jnp.int32)
  dst = edge_index[1].astype(jnp.int32)
  srcb = src.reshape(NBLK, B)
  dstb = dst.reshape(NBLK, B)
  out = _sc_scatter_mean(srcb, dstb, h[:, :DH], h[:, DH:])
  return out[:N]


# speedup vs baseline: 1.0401x; 1.0401x over previous
import functools

import jax
import jax.numpy as jnp
from jax import lax
from jax.experimental import pallas as pl
from jax.experimental.pallas import tpu as pltpu
from jax.experimental.pallas import tpu_sc as plsc

N = 10000
NPAD = 10240
D = 128
DH = 64
E = 320000
B = 125
NBLK = E // B
NSUB = 16
NB = NBLK // NSUB
IDXC = 80
NSLAB = NB // IDXC
ROWS_PER_TILE = NPAD // NSUB
CW = 16
DIV_CHUNK = 40
NBUF = 5
AHEAD = 4


def _sc_scatter_mean(srcb, dstb, h0, h1):
  mesh = plsc.VectorSubcoreMesh(core_axis_name="c", subcore_axis_name="s")

  @functools.partial(
      pl.kernel,
      out_type=jax.ShapeDtypeStruct((NPAD, D), jnp.float32),
      mesh=mesh,
      scratch_types=[
          pltpu.VMEM_SHARED((NPAD, DH), jnp.float32),
          pltpu.VMEM_SHARED((NPAD, CW), jnp.float32),
          pltpu.VMEM((IDXC, B), jnp.int32),
          pltpu.VMEM((IDXC, B), jnp.int32),
          pltpu.VMEM((NBUF, B, DH), jnp.float32),
          pltpu.VMEM((B, CW), jnp.float32),
          pltpu.VMEM((DIV_CHUNK, DH), jnp.float32),
          pltpu.VMEM((DIV_CHUNK, CW), jnp.float32),
          pltpu.SemaphoreType.DMA,
          pltpu.SemaphoreType.DMA,
          pltpu.SemaphoreType.DMA,
          pltpu.SemaphoreType.DMA,
          pltpu.SemaphoreType.DMA,
          pltpu.SemaphoreType.DMA,
          pltpu.SemaphoreType.DMA,
          pltpu.SemaphoreType.DMA,
          pltpu.SemaphoreType.DMA,
          pltpu.SemaphoreType.DMA,
          pltpu.SemaphoreType.DMA,
          pltpu.SemaphoreType.DMA,
          pltpu.SemaphoreType.DMA,
          pltpu.SemaphoreType.DMA,
          pltpu.SemaphoreType.DMA,
      ],
      compiler_params=pltpu.CompilerParams(use_tc_tiling_on_sc=False),
  )
  def k(srcb_hbm, dstb_hbm, h0_hbm, h1_hbm, out_hbm,
        acc, cnt, src_v, dst_v, rows, ones_v, accv, cntv,
        g0, g1, g2, g3, g4, s0, s1, s2, s3, s4, o0, o1, o2, o3, o4):
    c = lax.axis_index("c")
    s = lax.axis_index("s")
    row0 = s * ROWS_PER_TILE
    blk0 = s * NB
    gsem = [g0, g1, g2, g3, g4]
    ssem = [s0, s1, s2, s3, s4]
    osem = [o0, o1, o2, o3, o4]

    @pl.loop(0, B)
    def _(i):
      ones_v[i, :] = jnp.ones((CW,), jnp.float32)

    @pl.loop(0, DIV_CHUNK)
    def _(i):
      cntv[i, :] = jnp.zeros((CW,), jnp.float32)
      for q in range(DH // 16):
        accv[i, pl.ds(q * 16, 16)] = jnp.zeros((16,), jnp.float32)

    @pl.loop(0, ROWS_PER_TILE, step=DIV_CHUNK)
    def _(t):
      pltpu.sync_copy(accv, acc.at[pl.ds(row0 + t, DIV_CHUNK)])
      pltpu.sync_copy(cntv, cnt.at[pl.ds(row0 + t, DIV_CHUNK)])

    plsc.subcore_barrier()

    def gather(j, b):
      @pl.when(c == 0)
      def _():
        pltpu.async_copy(h0_hbm.at[src_v.at[j]], rows.at[b], gsem[b])

      @pl.when(c == 1)
      def _():
        pltpu.async_copy(h1_hbm.at[src_v.at[j]], rows.at[b], gsem[b])

    def gather_wait(j, b):
      pltpu.make_async_copy(h0_hbm.at[src_v.at[j]], rows.at[b], gsem[b]).wait()

    @pl.loop(0, NSLAB)
    def _(sl):
      sblk = blk0 + sl * IDXC
      pltpu.async_copy(srcb_hbm.at[pl.ds(sblk, IDXC)], src_v, g0)
      pltpu.async_copy(dstb_hbm.at[pl.ds(sblk, IDXC)], dst_v, g1)
      pltpu.make_async_copy(srcb_hbm.at[pl.ds(sblk, IDXC)], src_v, g0).wait()
      pltpu.make_async_copy(dstb_hbm.at[pl.ds(sblk, IDXC)], dst_v, g1).wait()

      for j in range(AHEAD):
        gather(j, j)

      @pl.loop(0, IDXC, step=NBUF)
      def _(i):
        for r in range(NBUF):
          b = r
          jj = i + r
          gather_wait(jj, b)
          pltpu.async_copy(rows.at[b], acc.at[dst_v.at[jj]], ssem[b], add=True)
          pltpu.async_copy(ones_v, cnt.at[dst_v.at[jj]], osem[b], add=True)

          @pl.when(jj >= NBUF - AHEAD)
          def _():
            bw = (r + AHEAD) % NBUF
            pltpu.make_async_copy(rows.at[bw],
                                  acc.at[dst_v.at[jj - (NBUF - AHEAD)]],
                                  ssem[bw]).wait()
            pltpu.make_async_copy(ones_v,
                                  cnt.at[dst_v.at[jj - (NBUF - AHEAD)]],
                                  osem[bw]).wait()

          @pl.when(jj + AHEAD < IDXC)
          def _():
            gather(jj + AHEAD, (r + AHEAD) % NBUF)

      for jj in range(IDXC - (NBUF - AHEAD), IDXC):
        b = jj % NBUF
        pltpu.make_async_copy(rows.at[b], acc.at[dst_v.at[jj]], ssem[b]).wait()
        pltpu.make_async_copy(ones_v, cnt.at[dst_v.at[jj]], osem[b]).wait()

    plsc.subcore_barrier()

    @pl.loop(0, ROWS_PER_TILE, step=DIV_CHUNK)
    def _(t):
      pltpu.sync_copy(acc.at[pl.ds(row0 + t, DIV_CHUNK)], accv)
      pltpu.sync_copy(cnt.at[pl.ds(row0 + t, DIV_CHUNK)], cntv)

      @pl.loop(0, DIV_CHUNK)
      def _(i):
        r = 1.0 / jnp.maximum(cntv[i, :], 1.0)
        for q in range(DH // 16):
          accv[i, pl.ds(q * 16, 16)] = accv[i, pl.ds(q * 16, 16)] * r

      pltpu.sync_copy(
          accv, out_hbm.at[pl.ds(row0 + t, DIV_CHUNK), pl.ds(c * DH, DH)])

  return k(srcb, dstb, h0, h1)


@jax.jit
def kernel(edge_index, h):
  src = edge_index[0].astype(jnp.int32)
  dst = edge_index[1].astype(jnp.int32)
  srcb = src.reshape(NBLK, B)
  dstb = dst.reshape(NBLK, B)
  out = _sc_scatter_mean(srcb, dstb, h[:, :DH], h[:, DH:])
  return out[:N]


# speedup vs baseline: 1.0661x; 1.0251x over previous
import functools

import jax
import jax.numpy as jnp
from jax import lax
from jax.experimental import pallas as pl
from jax.experimental.pallas import tpu as pltpu
from jax.experimental.pallas import tpu_sc as plsc

N = 10000
NPAD = 10240
D = 128
DH = 64
E = 320000
B = 125
NBLK = E // B
NSUB = 16
NB = NBLK // NSUB
IDXC = 80
NSLAB = NB // IDXC
ROWS_PER_TILE = NPAD // NSUB
CW = 16
DIV_CHUNK = 80
NBUF = 5
AHEAD = 4


def _sc_scatter_mean(srcb, dstb, h0, h1):
  mesh = plsc.VectorSubcoreMesh(core_axis_name="c", subcore_axis_name="s")

  @functools.partial(
      pl.kernel,
      out_type=jax.ShapeDtypeStruct((NPAD, D), jnp.float32),
      mesh=mesh,
      scratch_types=[
          pltpu.VMEM_SHARED((NPAD, DH), jnp.float32),
          pltpu.VMEM_SHARED((NPAD, CW), jnp.float32),
          pltpu.VMEM((IDXC, B), jnp.int32),
          pltpu.VMEM((IDXC, B), jnp.int32),
          pltpu.VMEM((NBUF, B, DH), jnp.float32),
          pltpu.VMEM((B, CW), jnp.float32),
          pltpu.VMEM((DIV_CHUNK, DH), jnp.float32),
          pltpu.VMEM((DIV_CHUNK, CW), jnp.float32),
          pltpu.SemaphoreType.DMA,
          pltpu.SemaphoreType.DMA,
          pltpu.SemaphoreType.DMA,
          pltpu.SemaphoreType.DMA,
          pltpu.SemaphoreType.DMA,
          pltpu.SemaphoreType.DMA,
          pltpu.SemaphoreType.DMA,
          pltpu.SemaphoreType.DMA,
          pltpu.SemaphoreType.DMA,
          pltpu.SemaphoreType.DMA,
          pltpu.SemaphoreType.DMA,
          pltpu.SemaphoreType.DMA,
          pltpu.SemaphoreType.DMA,
          pltpu.SemaphoreType.DMA,
          pltpu.SemaphoreType.DMA,
      ],
      compiler_params=pltpu.CompilerParams(use_tc_tiling_on_sc=False),
  )
  def k(srcb_hbm, dstb_hbm, h0_hbm, h1_hbm, out_hbm,
        acc, cnt, src_v, dst_v, rows, ones_v, accv, cntv,
        g0, g1, g2, g3, g4, s0, s1, s2, s3, s4, o0, o1, o2, o3, o4):
    c = lax.axis_index("c")
    s = lax.axis_index("s")
    row0 = s * ROWS_PER_TILE
    blk0 = s * NB
    gsem = [g0, g1, g2, g3, g4]
    ssem = [s0, s1, s2, s3, s4]
    osem = [o0, o1, o2, o3, o4]

    pltpu.async_copy(srcb_hbm.at[pl.ds(blk0, IDXC)], src_v, g0)
    pltpu.async_copy(dstb_hbm.at[pl.ds(blk0, IDXC)], dst_v, g1)

    @pl.loop(0, B)
    def _(i):
      ones_v[i, :] = jnp.ones((CW,), jnp.float32)

    @pl.loop(0, DIV_CHUNK)
    def _(i):
      cntv[i, :] = jnp.zeros((CW,), jnp.float32)
      for q in range(DH // 16):
        accv[i, pl.ds(q * 16, 16)] = jnp.zeros((16,), jnp.float32)

    @pl.loop(0, ROWS_PER_TILE, step=DIV_CHUNK)
    def _(t):
      pltpu.sync_copy(accv, acc.at[pl.ds(row0 + t, DIV_CHUNK)])
      pltpu.sync_copy(cntv, cnt.at[pl.ds(row0 + t, DIV_CHUNK)])

    plsc.subcore_barrier()

    def gather(j, b):
      @pl.when(c == 0)
      def _():
        pltpu.async_copy(h0_hbm.at[src_v.at[j]], rows.at[b], gsem[b])

      @pl.when(c == 1)
      def _():
        pltpu.async_copy(h1_hbm.at[src_v.at[j]], rows.at[b], gsem[b])

    def gather_wait(j, b):
      pltpu.make_async_copy(h0_hbm.at[src_v.at[j]], rows.at[b], gsem[b]).wait()

    @pl.loop(0, NSLAB)
    def _(sl):
      sblk = blk0 + sl * IDXC

      @pl.when(sl > 0)
      def _():
        pltpu.async_copy(srcb_hbm.at[pl.ds(sblk, IDXC)], src_v, g0)
        pltpu.async_copy(dstb_hbm.at[pl.ds(sblk, IDXC)], dst_v, g1)

      pltpu.make_async_copy(srcb_hbm.at[pl.ds(sblk, IDXC)], src_v, g0).wait()
      pltpu.make_async_copy(dstb_hbm.at[pl.ds(sblk, IDXC)], dst_v, g1).wait()

      for j in range(AHEAD):
        gather(j, j)

      @pl.loop(0, IDXC, step=NBUF)
      def _(i):
        for r in range(NBUF):
          b = r
          jj = i + r
          gather_wait(jj, b)
          pltpu.async_copy(rows.at[b], acc.at[dst_v.at[jj]], ssem[b], add=True)
          pltpu.async_copy(ones_v, cnt.at[dst_v.at[jj]], osem[b], add=True)

          @pl.when(jj >= NBUF - AHEAD)
          def _():
            bw = (r + AHEAD) % NBUF
            pltpu.make_async_copy(rows.at[bw],
                                  acc.at[dst_v.at[jj - (NBUF - AHEAD)]],
                                  ssem[bw]).wait()
            pltpu.make_async_copy(ones_v,
                                  cnt.at[dst_v.at[jj - (NBUF - AHEAD)]],
                                  osem[bw]).wait()

          @pl.when(jj + AHEAD < IDXC)
          def _():
            gather(jj + AHEAD, (r + AHEAD) % NBUF)

      for jj in range(IDXC - (NBUF - AHEAD), IDXC):
        b = jj % NBUF
        pltpu.make_async_copy(rows.at[b], acc.at[dst_v.at[jj]], ssem[b]).wait()
        pltpu.make_async_copy(ones_v, cnt.at[dst_v.at[jj]], osem[b]).wait()

    plsc.subcore_barrier()

    @pl.loop(0, ROWS_PER_TILE, step=DIV_CHUNK)
    def _(t):
      pltpu.sync_copy(acc.at[pl.ds(row0 + t, DIV_CHUNK)], accv)
      pltpu.sync_copy(cnt.at[pl.ds(row0 + t, DIV_CHUNK)], cntv)

      @pl.loop(0, DIV_CHUNK)
      def _(i):
        r = 1.0 / jnp.maximum(cntv[i, :], 1.0)
        for q in range(DH // 16):
          accv[i, pl.ds(q * 16, 16)] = accv[i, pl.ds(q * 16, 16)] * r

      pltpu.sync_copy(
          accv, out_hbm.at[pl.ds(row0 + t, DIV_CHUNK), pl.ds(c * DH, DH)])

  return k(srcb, dstb, h0, h1)


@jax.jit
def kernel(edge_index, h):
  src = edge_index[0].astype(jnp.int32)
  dst = edge_index[1].astype(jnp.int32)
  srcb = src.reshape(NBLK, B)
  dstb = dst.reshape(NBLK, B)
  out = _sc_scatter_mean(srcb, dstb, h[:, :DH], h[:, DH:])
  return out[:N]
